# baseline (speedup 1.0000x reference)
import functools

import jax
import jax.numpy as jnp
from jax import lax
from jax.experimental import pallas as pl
from jax.experimental.pallas import tpu as pltpu
from jax.experimental.pallas import tpu_sc as plsc

B, S, D = 2, 2048, 1024
E, K, R = 16, 2, 16
ER = E * R
T = B * S
SCALE = 1.0

TS = 2048


def _tc_fwd_body(x_ref, wr_ref, logits_ref):
    lg = jnp.dot(x_ref[...], wr_ref[...], preferred_element_type=jnp.float32)
    logits_ref[...] = jnp.concatenate(
        [lg, jnp.zeros((lg.shape[0], 128 - E), jnp.float32)], axis=1
    )


def _tc_fwd(x2d, w_router):
    return pl.pallas_call(
        _tc_fwd_body,
        grid=(T // TS,),
        in_specs=[
            pl.BlockSpec((TS, D), lambda i: (i, 0)),
            pl.BlockSpec((D, E), lambda i: (0, 0)),
        ],
        out_specs=pl.BlockSpec((TS, 128), lambda i: (i, 0)),
        out_shape=jax.ShapeDtypeStruct((T, 128), jnp.float32),
        compiler_params=pltpu.CompilerParams(
            dimension_semantics=("arbitrary",),
        ),
    )(x2d, w_router)


def _make_sc_gather(n_rows):
    info = plsc.get_sparse_core_info()
    nw = info.num_subcores
    chunk = n_rows // nw

    mesh = plsc.VectorSubcoreMesh(
        core_axis_name="c", subcore_axis_name="s", num_cores=1
    )

    @functools.partial(
        pl.kernel,
        mesh=mesh,
        out_type=jax.ShapeDtypeStruct((n_rows, 128), jnp.float32),
        compiler_params=pltpu.CompilerParams(skip_device_barrier=True),
        scratch_types=[
            pltpu.VMEM((chunk,), jnp.int32),
            pltpu.VMEM((chunk, 128), jnp.float32),
            pltpu.SemaphoreType.DMA,
        ],
    )
    def sc_gather(logits_hbm, gidx_hbm, rl_out, idx_v, rows_v, sem):
        base = lax.axis_index("s") * chunk

        pltpu.sync_copy(gidx_hbm.at[pl.ds(base, chunk)], idx_v)
        pltpu.async_copy(logits_hbm.at[idx_v], rows_v, sem).wait()
        pltpu.sync_copy(rows_v, rl_out.at[pl.ds(base, chunk)])

    return sc_gather


_sc_gather_half = _make_sc_gather(S)


def _tc_out_body(x_ref, rl_ref, af_ref, bf_ref, *rest):
    o_ref, rl_out_ref = rest[-2:]
    rl = rl_ref[:, :E]
    lanes = lax.broadcasted_iota(jnp.int32, rl.shape, 1)
    m1 = jnp.max(rl, axis=-1, keepdims=True)
    i1 = jnp.min(jnp.where(rl == m1, lanes, E), axis=-1, keepdims=True)
    sel1 = lanes == i1
    rl2 = jnp.where(sel1, -jnp.inf, rl)
    m2 = jnp.max(rl2, axis=-1, keepdims=True)
    i2 = jnp.min(jnp.where(rl2 == m2, lanes, E), axis=-1, keepdims=True)
    sel2 = lanes == i2
    d = jnp.exp(m2 - m1)
    w1 = 1.0 / (1.0 + d)
    w2 = d * w1
    mask = jnp.where(sel1, w1, 0.0) + jnp.where(sel2, w2, 0.0)

    row = lax.broadcasted_iota(jnp.int32, (E, ER), 0)
    col = lax.broadcasted_iota(jnp.int32, (E, ER), 1)
    expand = (col // R == row).astype(jnp.float32)
    me = jnp.dot(mask, expand, preferred_element_type=jnp.float32)

    x = x_ref[...]
    h = jnp.dot(
        x.astype(jnp.bfloat16), af_ref[...], preferred_element_type=jnp.float32
    )
    hm = (h * me).astype(jnp.bfloat16)
    o_ref[...] = x + SCALE * jnp.dot(
        hm, bf_ref[...], preferred_element_type=jnp.float32
    )
    rl_out_ref[...] = rl


TS2 = 1024
_HB = S // TS2


def _tc_out_half(x2d, rl_half, a_flat, b_flat, half, prev=None):
    off = half * _HB
    in_specs = [
        pl.BlockSpec((TS2, D), lambda i: (i + off, 0)),
        pl.BlockSpec((TS2, 128), lambda i: (i, 0)),
        pl.BlockSpec((D, ER), lambda i: (0, 0)),
        pl.BlockSpec((ER, D), lambda i: (0, 0)),
    ]
    operands = [x2d, rl_half, a_flat, b_flat]
    kwargs = {}
    if prev is not None:
        in_specs += [
            pl.BlockSpec(memory_space=pl.ANY),
            pl.BlockSpec(memory_space=pl.ANY),
        ]
        operands += list(prev)
        kwargs["input_output_aliases"] = {4: 0, 5: 1}
    return pl.pallas_call(
        _tc_out_body,
        grid=(_HB,),
        in_specs=in_specs,
        out_specs=[
            pl.BlockSpec((TS2, D), lambda i: (i + off, 0)),
            pl.BlockSpec((TS2, E), lambda i: (i + off, 0)),
        ],
        out_shape=[
            jax.ShapeDtypeStruct((T, D), jnp.float32),
            jax.ShapeDtypeStruct((T, E), jnp.float32),
        ],
        compiler_params=pltpu.CompilerParams(
            dimension_semantics=("arbitrary",),
        ),
        **kwargs,
    )(*operands)


def kernel(x, router_hs_idxs, W_router, A, Bw):
    x2d = x.reshape(T, D)
    a_flat = jnp.transpose(A.astype(jnp.bfloat16), (1, 0, 2)).reshape(D, ER)
    b_flat = Bw.astype(jnp.bfloat16).reshape(ER, D)
    gidx = (router_hs_idxs + jnp.arange(B, dtype=jnp.int32)[:, None] * S).reshape(T)

    logits_all = _tc_fwd(x2d, W_router)
    rl0 = _sc_gather_half(logits_all, gidx[:S])
    rl1 = _sc_gather_half(logits_all, gidx[S:])
    prev = _tc_out_half(x2d, rl0, a_flat, b_flat, 0)
    out, router_logits = _tc_out_half(x2d, rl1, a_flat, b_flat, 1, prev=prev)

    return out.reshape(B, S, D), router_logits.reshape(B, S, E)


# speedup vs baseline: 1.0455x; 1.0455x over previous
import functools

import jax
import jax.numpy as jnp
from jax import lax
from jax.experimental import pallas as pl
from jax.experimental.pallas import tpu as pltpu
from jax.experimental.pallas import tpu_sc as plsc

B, S, D = 2, 2048, 1024
E, K, R = 16, 2, 16
ER = E * R
T = B * S
SCALE = 1.0

TS = 2048


def _tc_fwd_body(x_ref, wr_ref, logits_ref):
    lg = jnp.dot(x_ref[...], wr_ref[...], preferred_element_type=jnp.float32)
    logits_ref[...] = jnp.concatenate(
        [lg, jnp.zeros((lg.shape[0], 128 - E), jnp.float32)], axis=1
    )


def _tc_fwd(x2d, w_router):
    return pl.pallas_call(
        _tc_fwd_body,
        grid=(T // TS,),
        in_specs=[
            pl.BlockSpec((TS, D), lambda i: (i, 0)),
            pl.BlockSpec((D, E), lambda i: (0, 0)),
        ],
        out_specs=pl.BlockSpec((TS, 128), lambda i: (i, 0)),
        out_shape=jax.ShapeDtypeStruct((T, 128), jnp.float32),
        compiler_params=pltpu.CompilerParams(
            dimension_semantics=("arbitrary",),
        ),
    )(x2d, w_router)


def _make_sc_gather():
    info = plsc.get_sparse_core_info()
    nw = info.num_subcores
    chunk = T // nw

    mesh = plsc.VectorSubcoreMesh(
        core_axis_name="c", subcore_axis_name="s", num_cores=1
    )

    @functools.partial(
        pl.kernel,
        mesh=mesh,
        out_type=jax.ShapeDtypeStruct((T, 128), jnp.float32),
        compiler_params=pltpu.CompilerParams(skip_device_barrier=True),
        scratch_types=[
            pltpu.VMEM((chunk,), jnp.int32),
            pltpu.VMEM((chunk, 128), jnp.float32),
            pltpu.SemaphoreType.DMA,
        ],
    )
    def sc_gather(logits_hbm, gidx_hbm, rl_out, idx_v, rows_v, sem):
        base = lax.axis_index("s") * chunk

        pltpu.sync_copy(gidx_hbm.at[pl.ds(base, chunk)], idx_v)
        pltpu.async_copy(logits_hbm.at[idx_v], rows_v, sem).wait()
        pltpu.sync_copy(rows_v, rl_out.at[pl.ds(base, chunk)])

    return sc_gather


_sc_gather = _make_sc_gather()


def _tc_out_body(x_ref, rl_ref, af_ref, bf_ref, o_ref, rl_out_ref):
    rl = rl_ref[:, :E]
    lanes = lax.broadcasted_iota(jnp.int32, rl.shape, 1)
    m1 = jnp.max(rl, axis=-1, keepdims=True)
    i1 = jnp.min(jnp.where(rl == m1, lanes, E), axis=-1, keepdims=True)
    sel1 = lanes == i1
    rl2 = jnp.where(sel1, -jnp.inf, rl)
    m2 = jnp.max(rl2, axis=-1, keepdims=True)
    i2 = jnp.min(jnp.where(rl2 == m2, lanes, E), axis=-1, keepdims=True)
    sel2 = lanes == i2
    d = jnp.exp(m2 - m1)
    w1 = 1.0 / (1.0 + d)
    w2 = d * w1
    mask = jnp.where(sel1, w1, 0.0) + jnp.where(sel2, w2, 0.0)

    row = lax.broadcasted_iota(jnp.int32, (E, ER), 0)
    col = lax.broadcasted_iota(jnp.int32, (E, ER), 1)
    expand = (col // R == row).astype(jnp.float32)
    me = jnp.dot(mask, expand, preferred_element_type=jnp.float32)

    x = x_ref[...]
    h = jnp.dot(
        x.astype(jnp.bfloat16), af_ref[...], preferred_element_type=jnp.float32
    )
    hm = (h * me).astype(jnp.bfloat16)
    o_ref[...] = x + SCALE * jnp.dot(
        hm, bf_ref[...], preferred_element_type=jnp.float32
    )
    rl_out_ref[...] = rl


def _tc_out(x2d, router_logits, a_flat, b_flat):
    return pl.pallas_call(
        _tc_out_body,
        grid=(T // TS,),
        in_specs=[
            pl.BlockSpec((TS, D), lambda i: (i, 0)),
            pl.BlockSpec((TS, 128), lambda i: (i, 0)),
            pl.BlockSpec((D, ER), lambda i: (0, 0)),
            pl.BlockSpec((ER, D), lambda i: (0, 0)),
        ],
        out_specs=[
            pl.BlockSpec((TS, D), lambda i: (i, 0)),
            pl.BlockSpec((TS, E), lambda i: (i, 0)),
        ],
        out_shape=[
            jax.ShapeDtypeStruct((T, D), jnp.float32),
            jax.ShapeDtypeStruct((T, E), jnp.float32),
        ],
        compiler_params=pltpu.CompilerParams(
            dimension_semantics=("arbitrary",),
        ),
    )(x2d, router_logits, a_flat, b_flat)


def kernel(x, router_hs_idxs, W_router, A, Bw):
    x2d = x.reshape(T, D)
    a_flat = jnp.transpose(A.astype(jnp.bfloat16), (1, 0, 2)).reshape(D, ER)
    b_flat = Bw.astype(jnp.bfloat16).reshape(ER, D)
    gidx = (router_hs_idxs + jnp.arange(B, dtype=jnp.int32)[:, None] * S).reshape(T)

    logits_all = _tc_fwd(x2d, W_router)
    rl_pad = _sc_gather(logits_all, gidx)
    out, router_logits = _tc_out(x2d, rl_pad, a_flat, b_flat)

    return out.reshape(B, S, D), router_logits.reshape(B, S, E)
